# baseline (speedup 1.0000x reference)
import functools

import jax
import jax.numpy as jnp
from jax.experimental import pallas as pl
from jax.experimental.pallas import tpu as pltpu

BATCH = 4096
TILE = 512
INPUT_DIM = 768
EMBED_DIM = 64
CODEBOOK_SIZE = 1024
N_LAYERS = 3
N_CAT = 18
COMMIT_W = 0.25


def _silu(v):
    return 0.5 * v * (1.0 + jnp.tanh(0.5 * v))


def _dot(a, b):
    return jax.lax.dot_general(a, b, (((1,), (0,)), ((), ())),
                               preferred_element_type=jnp.float32)


def _bdot(a, b):
    return jax.lax.dot_general(a.astype(jnp.bfloat16), b,
                               (((1,), (0,)), ((), ())),
                               preferred_element_type=jnp.float32)


def _fused_body(x_ref,
                ew0, eb0, ew1, eb1, ew2, eb2, ew3, eb3,
                dw0, db0, dw1, db1, dw2, db2, dw3, db3,
                cb0, cb1, cb2,
                recon_ref, rq_ref):
    i = pl.program_id(0)
    xb = x_ref[...]

    h = _silu(_bdot(xb, ew0[...]) + eb0[...])
    h = _silu(_bdot(h, ew1[...]) + eb1[...])
    h = _silu(_bdot(h, ew2[...]) + eb2[...])
    h = _bdot(h, ew3[...]) + eb3[...]
    n = jnp.sqrt(jnp.sum(h * h, axis=-1, keepdims=True))
    res = h / jnp.maximum(n, 1e-12)

    col1024 = jax.lax.broadcasted_iota(jnp.int32, (TILE, CODEBOOK_SIZE), 1)
    emb_sum = jnp.zeros((TILE, EMBED_DIM), dtype=jnp.float32)
    rq_row = jnp.zeros((TILE, 1), dtype=jnp.float32)

    for cb_ref in (cb0, cb1, cb2):
        cbm = cb_ref[...]
        cbn = jnp.sqrt(jnp.sum(cbm * cbm, axis=-1, keepdims=True))
        cb = cbm / jnp.maximum(cbn, 1e-12)
        res_sq = jnp.sum(res * res, axis=-1, keepdims=True)
        cb_sq = jnp.sum(cb * cb, axis=-1)[None, :]
        dist = res_sq + cb_sq - 2.0 * _dot(res, cb.T)
        dmin = jnp.min(dist, axis=-1, keepdims=True)
        idx = jnp.min(jnp.where(dist == dmin, col1024, CODEBOOK_SIZE),
                      axis=-1, keepdims=True)
        onehot = (col1024 == idx).astype(jnp.float32)
        emb = _dot(onehot, cb)
        rn = jnp.sqrt(jnp.sum(res * res, axis=-1, keepdims=True))
        u = res / (rn + 1e-8)
        qn = jnp.sqrt(jnp.sum(emb * emb, axis=-1, keepdims=True))
        q = emb / (qn + 1e-8)
        w = u + q
        wn = jnp.sqrt(jnp.sum(w * w, axis=-1, keepdims=True))
        w = w / jnp.maximum(wn, 1e-6)
        ew = jnp.sum(res * w, axis=-1, keepdims=True)
        eu = jnp.sum(res * u, axis=-1, keepdims=True)
        out = res - 2.0 * ew * w + 2.0 * eu * q
        new_res = res - out
        rq_row = rq_row + jnp.sum(new_res * new_res, axis=-1, keepdims=True)
        emb_sum = emb_sum + out
        res = new_res

    h = _silu(_bdot(emb_sum, dw0[...]) + db0[...])
    h = _silu(_bdot(h, dw1[...]) + db1[...])
    h = _silu(_bdot(h, dw2[...]) + db2[...])
    x_hat = _bdot(h, dw3[...]) + db3[...]

    colx = jax.lax.broadcasted_iota(jnp.int32, (TILE, INPUT_DIM), 1)
    is_mse = colx < (INPUT_DIM - N_CAT)
    diff = x_hat - xb
    mse_row = jnp.sum(jnp.where(is_mse, diff * diff, 0.0), axis=-1)
    lg = x_hat[:, INPUT_DIM - 128:]
    tg = xb[:, INPUT_DIM - 128:]
    colb = jax.lax.broadcasted_iota(jnp.int32, (TILE, 128), 1)
    bce = (jnp.maximum(lg, 0.0) - lg * tg
           + jnp.log1p(jnp.exp(-jnp.abs(lg))))
    bce_row = jnp.sum(jnp.where(colb >= 128 - N_CAT, bce, 0.0), axis=-1)
    recon_sum = jnp.sum(mse_row + bce_row)
    rq_sum = (1.0 + COMMIT_W) * jnp.sum(rq_row)

    @pl.when(i == 0)
    def _init():
        recon_ref[...] = jnp.zeros((1, 1), jnp.float32)
        rq_ref[...] = jnp.zeros((1, 1), jnp.float32)

    recon_ref[...] += recon_sum.reshape(1, 1)
    rq_ref[...] += rq_sum.reshape(1, 1)


@functools.partial(jax.jit, static_argnames=())
def _fused(x, ew0, eb0, ew1, eb1, ew2, eb2, ew3, eb3,
           dw0, db0, dw1, db1, dw2, db2, dw3, db3, cb0, cb1, cb2):
    num_tiles = BATCH // TILE

    def wspec(shape):
        return pl.BlockSpec(shape, lambda i: (0,) * len(shape))

    in_specs = [pl.BlockSpec((TILE, INPUT_DIM), lambda i: (i, 0))]
    ws = [ew0, eb0, ew1, eb1, ew2, eb2, ew3, eb3,
          dw0, db0, dw1, db1, dw2, db2, dw3, db3, cb0, cb1, cb2]
    in_specs += [wspec(w.shape) for w in ws]

    out_shape = (jax.ShapeDtypeStruct((1, 1), jnp.float32),
                 jax.ShapeDtypeStruct((1, 1), jnp.float32))
    out_specs = (pl.BlockSpec((1, 1), lambda i: (0, 0)),
                 pl.BlockSpec((1, 1), lambda i: (0, 0)))

    recon_sum, rq_sum = pl.pallas_call(
        _fused_body,
        grid=(num_tiles,),
        in_specs=in_specs,
        out_specs=out_specs,
        out_shape=out_shape,
        compiler_params=pltpu.CompilerParams(
            dimension_semantics=("arbitrary",),
            vmem_limit_bytes=128 * 1024 * 1024,
        ),
    )(x, *ws)
    return recon_sum[0, 0], rq_sum[0, 0]


def kernel(x, enc_w0, enc_b0, enc_w1, enc_b1, enc_w2, enc_b2, enc_w3, enc_b3,
           dec_w0, dec_b0, dec_w1, dec_b1, dec_w2, dec_b2, dec_w3, dec_b3,
           cb0, cb1, cb2, gumbel_t):
    del gumbel_t
    bs = [b.reshape(1, -1) for b in
          (enc_b0, enc_b1, enc_b2, enc_b3, dec_b0, dec_b1, dec_b2, dec_b3)]
    ws = [w.astype(jnp.bfloat16) for w in
          (enc_w0, enc_w1, enc_w2, enc_w3, dec_w0, dec_w1, dec_w2, dec_w3)]
    recon_sum, rq_sum = _fused(
        x, ws[0], bs[0], ws[1], bs[1], ws[2], bs[2], ws[3], bs[3],
        ws[4], bs[4], ws[5], bs[5], ws[6], bs[6], ws[7], bs[7],
        cb0, cb1, cb2)
    recon_mean = recon_sum / BATCH
    rq_mean = rq_sum / BATCH
    loss = recon_mean + 3.0 * rq_mean
    return loss, recon_mean, rq_mean


# speedup vs baseline: 1.0866x; 1.0866x over previous
import functools

import jax
import jax.numpy as jnp
from jax.experimental import pallas as pl
from jax.experimental.pallas import tpu as pltpu

BATCH = 4096
TILE = 512
INPUT_DIM = 768
EMBED_DIM = 64
CODEBOOK_SIZE = 1024
N_LAYERS = 3
N_CAT = 18
COMMIT_W = 0.25


def _silu(v):
    return 0.5 * v * (1.0 + jnp.tanh(0.5 * v))


def _dot(a, b):
    return jax.lax.dot_general(a, b, (((1,), (0,)), ((), ())),
                               preferred_element_type=jnp.float32)


def _fused_body(x_ref,
                ew0, eb0, ew1, eb1, ew2, eb2, ew3, eb3,
                dw0, db0, dw1, db1, dw2, db2, dw3, db3,
                cb0, cb1, cb2,
                recon_ref, rq_ref):
    i = pl.program_id(0)
    xb = x_ref[...]

    h = _silu(_dot(xb, ew0[...]) + eb0[...])
    h = _silu(_dot(h, ew1[...]) + eb1[...])
    h = _silu(_dot(h, ew2[...]) + eb2[...])
    h = _dot(h, ew3[...]) + eb3[...]
    n = jnp.sqrt(jnp.sum(h * h, axis=-1, keepdims=True))
    res = h / jnp.maximum(n, 1e-12)

    col1024 = jax.lax.broadcasted_iota(jnp.int32, (TILE, CODEBOOK_SIZE), 1)
    emb_sum = jnp.zeros((TILE, EMBED_DIM), dtype=jnp.float32)
    rq_row = jnp.zeros((TILE, 1), dtype=jnp.float32)

    for cb_ref in (cb0, cb1, cb2):
        cbm = cb_ref[...]
        cbn = jnp.sqrt(jnp.sum(cbm * cbm, axis=-1, keepdims=True))
        cb = cbm / jnp.maximum(cbn, 1e-12)
        res_sq = jnp.sum(res * res, axis=-1, keepdims=True)
        cb_sq = jnp.sum(cb * cb, axis=-1)[None, :]
        dist = res_sq + cb_sq - 2.0 * _dot(res, cb.T)
        dmin = jnp.min(dist, axis=-1, keepdims=True)
        idx = jnp.min(jnp.where(dist == dmin, col1024, CODEBOOK_SIZE),
                      axis=-1, keepdims=True)
        onehot = (col1024 == idx).astype(jnp.float32)
        emb = _dot(onehot, cb)
        rn = jnp.sqrt(jnp.sum(res * res, axis=-1, keepdims=True))
        u = res / (rn + 1e-8)
        qn = jnp.sqrt(jnp.sum(emb * emb, axis=-1, keepdims=True))
        q = emb / (qn + 1e-8)
        w = u + q
        wn = jnp.sqrt(jnp.sum(w * w, axis=-1, keepdims=True))
        w = w / jnp.maximum(wn, 1e-6)
        ew = jnp.sum(res * w, axis=-1, keepdims=True)
        eu = jnp.sum(res * u, axis=-1, keepdims=True)
        out = res - 2.0 * ew * w + 2.0 * eu * q
        new_res = res - out
        rq_row = rq_row + jnp.sum(new_res * new_res, axis=-1, keepdims=True)
        emb_sum = emb_sum + out
        res = new_res

    h = _silu(_dot(emb_sum, dw0[...]) + db0[...])
    h = _silu(_dot(h, dw1[...]) + db1[...])
    h = _silu(_dot(h, dw2[...]) + db2[...])
    x_hat = _dot(h, dw3[...]) + db3[...]

    colx = jax.lax.broadcasted_iota(jnp.int32, (TILE, INPUT_DIM), 1)
    is_mse = colx < (INPUT_DIM - N_CAT)
    diff = x_hat - xb
    mse_row = jnp.sum(jnp.where(is_mse, diff * diff, 0.0), axis=-1)
    lg = x_hat[:, INPUT_DIM - 128:]
    tg = xb[:, INPUT_DIM - 128:]
    colb = jax.lax.broadcasted_iota(jnp.int32, (TILE, 128), 1)
    bce = (jnp.maximum(lg, 0.0) - lg * tg
           + jnp.log1p(jnp.exp(-jnp.abs(lg))))
    bce_row = jnp.sum(jnp.where(colb >= 128 - N_CAT, bce, 0.0), axis=-1)
    recon_sum = jnp.sum(mse_row + bce_row)
    rq_sum = (1.0 + COMMIT_W) * jnp.sum(rq_row)

    @pl.when(i == 0)
    def _init():
        recon_ref[...] = jnp.zeros((1, 1), jnp.float32)
        rq_ref[...] = jnp.zeros((1, 1), jnp.float32)

    recon_ref[...] += recon_sum.reshape(1, 1)
    rq_ref[...] += rq_sum.reshape(1, 1)


@functools.partial(jax.jit, static_argnames=())
def _fused(x, ew0, eb0, ew1, eb1, ew2, eb2, ew3, eb3,
           dw0, db0, dw1, db1, dw2, db2, dw3, db3, cb0, cb1, cb2):
    num_tiles = BATCH // TILE

    def wspec(shape):
        return pl.BlockSpec(shape, lambda i: (0,) * len(shape))

    in_specs = [pl.BlockSpec((TILE, INPUT_DIM), lambda i: (i, 0))]
    ws = [ew0, eb0, ew1, eb1, ew2, eb2, ew3, eb3,
          dw0, db0, dw1, db1, dw2, db2, dw3, db3, cb0, cb1, cb2]
    in_specs += [wspec(w.shape) for w in ws]

    out_shape = (jax.ShapeDtypeStruct((1, 1), jnp.float32),
                 jax.ShapeDtypeStruct((1, 1), jnp.float32))
    out_specs = (pl.BlockSpec((1, 1), lambda i: (0, 0)),
                 pl.BlockSpec((1, 1), lambda i: (0, 0)))

    recon_sum, rq_sum = pl.pallas_call(
        _fused_body,
        grid=(num_tiles,),
        in_specs=in_specs,
        out_specs=out_specs,
        out_shape=out_shape,
        compiler_params=pltpu.CompilerParams(
            dimension_semantics=("arbitrary",),
            vmem_limit_bytes=128 * 1024 * 1024,
        ),
    )(x, *ws)
    return recon_sum[0, 0], rq_sum[0, 0]


def kernel(x, enc_w0, enc_b0, enc_w1, enc_b1, enc_w2, enc_b2, enc_w3, enc_b3,
           dec_w0, dec_b0, dec_w1, dec_b1, dec_w2, dec_b2, dec_w3, dec_b3,
           cb0, cb1, cb2, gumbel_t):
    del gumbel_t
    bs = [b.reshape(1, -1) for b in
          (enc_b0, enc_b1, enc_b2, enc_b3, dec_b0, dec_b1, dec_b2, dec_b3)]
    ws = [enc_w0, enc_w1, enc_w2, enc_w3, dec_w0, dec_w1, dec_w2, dec_w3]
    recon_sum, rq_sum = _fused(
        x, ws[0], bs[0], ws[1], bs[1], ws[2], bs[2], ws[3], bs[3],
        ws[4], bs[4], ws[5], bs[5], ws[6], bs[6], ws[7], bs[7],
        cb0, cb1, cb2)
    recon_mean = recon_sum / BATCH
    rq_mean = rq_sum / BATCH
    loss = recon_mean + 3.0 * rq_mean
    return loss, recon_mean, rq_mean


# speedup vs baseline: 1.1550x; 1.0629x over previous
import functools

import jax
import jax.numpy as jnp
from jax.experimental import pallas as pl
from jax.experimental.pallas import tpu as pltpu

BATCH = 4096
TILE = 1024
INPUT_DIM = 768
EMBED_DIM = 64
CODEBOOK_SIZE = 1024
N_LAYERS = 3
N_CAT = 18
COMMIT_W = 0.25


def _silu(v):
    return 0.5 * v * (1.0 + jnp.tanh(0.5 * v))


def _dot(a, b):
    return jax.lax.dot_general(a, b, (((1,), (0,)), ((), ())),
                               preferred_element_type=jnp.float32)


def _fused_body(x_ref,
                ew0, eb0, ew1, eb1, ew2, eb2, ew3, eb3,
                dw0, db0, dw1, db1, dw2, db2, dw3, db3,
                cb0, cb1, cb2,
                recon_ref, rq_ref):
    i = pl.program_id(0)
    xb = x_ref[...]

    h = _silu(_dot(xb, ew0[...]) + eb0[...])
    h = _silu(_dot(h, ew1[...]) + eb1[...])
    h = _silu(_dot(h, ew2[...]) + eb2[...])
    h = _dot(h, ew3[...]) + eb3[...]
    n = jnp.sqrt(jnp.sum(h * h, axis=-1, keepdims=True))
    res = h / jnp.maximum(n, 1e-12)

    col1024 = jax.lax.broadcasted_iota(jnp.int32, (TILE, CODEBOOK_SIZE), 1)
    emb_sum = jnp.zeros((TILE, EMBED_DIM), dtype=jnp.float32)
    rq_row = jnp.zeros((TILE, 1), dtype=jnp.float32)

    for cb_ref in (cb0, cb1, cb2):
        cbm = cb_ref[...]
        cbn = jnp.sqrt(jnp.sum(cbm * cbm, axis=-1, keepdims=True))
        cb = cbm / jnp.maximum(cbn, 1e-12)
        res_sq = jnp.sum(res * res, axis=-1, keepdims=True)
        cb_sq = jnp.sum(cb * cb, axis=-1)[None, :]
        dist = res_sq + cb_sq - 2.0 * _dot(res, cb.T)
        dmin = jnp.min(dist, axis=-1, keepdims=True)
        idx = jnp.min(jnp.where(dist == dmin, col1024, CODEBOOK_SIZE),
                      axis=-1, keepdims=True)
        onehot = (col1024 == idx).astype(jnp.float32)
        emb = _dot(onehot, cb)
        rn = jnp.sqrt(jnp.sum(res * res, axis=-1, keepdims=True))
        u = res / (rn + 1e-8)
        qn = jnp.sqrt(jnp.sum(emb * emb, axis=-1, keepdims=True))
        q = emb / (qn + 1e-8)
        w = u + q
        wn = jnp.sqrt(jnp.sum(w * w, axis=-1, keepdims=True))
        w = w / jnp.maximum(wn, 1e-6)
        ew = jnp.sum(res * w, axis=-1, keepdims=True)
        eu = jnp.sum(res * u, axis=-1, keepdims=True)
        out = res - 2.0 * ew * w + 2.0 * eu * q
        new_res = res - out
        rq_row = rq_row + jnp.sum(new_res * new_res, axis=-1, keepdims=True)
        emb_sum = emb_sum + out
        res = new_res

    h = _silu(_dot(emb_sum, dw0[...]) + db0[...])
    h = _silu(_dot(h, dw1[...]) + db1[...])
    h = _silu(_dot(h, dw2[...]) + db2[...])
    x_hat = _dot(h, dw3[...]) + db3[...]

    colx = jax.lax.broadcasted_iota(jnp.int32, (TILE, INPUT_DIM), 1)
    is_mse = colx < (INPUT_DIM - N_CAT)
    diff = x_hat - xb
    mse_row = jnp.sum(jnp.where(is_mse, diff * diff, 0.0), axis=-1)
    lg = x_hat[:, INPUT_DIM - 128:]
    tg = xb[:, INPUT_DIM - 128:]
    colb = jax.lax.broadcasted_iota(jnp.int32, (TILE, 128), 1)
    bce = (jnp.maximum(lg, 0.0) - lg * tg
           + jnp.log1p(jnp.exp(-jnp.abs(lg))))
    bce_row = jnp.sum(jnp.where(colb >= 128 - N_CAT, bce, 0.0), axis=-1)
    recon_sum = jnp.sum(mse_row + bce_row)
    rq_sum = (1.0 + COMMIT_W) * jnp.sum(rq_row)

    @pl.when(i == 0)
    def _init():
        recon_ref[...] = jnp.zeros((1, 1), jnp.float32)
        rq_ref[...] = jnp.zeros((1, 1), jnp.float32)

    recon_ref[...] += recon_sum.reshape(1, 1)
    rq_ref[...] += rq_sum.reshape(1, 1)


@functools.partial(jax.jit, static_argnames=())
def _fused(x, ew0, eb0, ew1, eb1, ew2, eb2, ew3, eb3,
           dw0, db0, dw1, db1, dw2, db2, dw3, db3, cb0, cb1, cb2):
    num_tiles = BATCH // TILE

    def wspec(shape):
        return pl.BlockSpec(shape, lambda i: (0,) * len(shape))

    in_specs = [pl.BlockSpec((TILE, INPUT_DIM), lambda i: (i, 0))]
    ws = [ew0, eb0, ew1, eb1, ew2, eb2, ew3, eb3,
          dw0, db0, dw1, db1, dw2, db2, dw3, db3, cb0, cb1, cb2]
    in_specs += [wspec(w.shape) for w in ws]

    out_shape = (jax.ShapeDtypeStruct((1, 1), jnp.float32),
                 jax.ShapeDtypeStruct((1, 1), jnp.float32))
    out_specs = (pl.BlockSpec((1, 1), lambda i: (0, 0)),
                 pl.BlockSpec((1, 1), lambda i: (0, 0)))

    recon_sum, rq_sum = pl.pallas_call(
        _fused_body,
        grid=(num_tiles,),
        in_specs=in_specs,
        out_specs=out_specs,
        out_shape=out_shape,
        compiler_params=pltpu.CompilerParams(
            dimension_semantics=("arbitrary",),
            vmem_limit_bytes=128 * 1024 * 1024,
        ),
    )(x, *ws)
    return recon_sum[0, 0], rq_sum[0, 0]


def kernel(x, enc_w0, enc_b0, enc_w1, enc_b1, enc_w2, enc_b2, enc_w3, enc_b3,
           dec_w0, dec_b0, dec_w1, dec_b1, dec_w2, dec_b2, dec_w3, dec_b3,
           cb0, cb1, cb2, gumbel_t):
    del gumbel_t
    bs = [b.reshape(1, -1) for b in
          (enc_b0, enc_b1, enc_b2, enc_b3, dec_b0, dec_b1, dec_b2, dec_b3)]
    ws = [enc_w0, enc_w1, enc_w2, enc_w3, dec_w0, dec_w1, dec_w2, dec_w3]
    recon_sum, rq_sum = _fused(
        x, ws[0], bs[0], ws[1], bs[1], ws[2], bs[2], ws[3], bs[3],
        ws[4], bs[4], ws[5], bs[5], ws[6], bs[6], ws[7], bs[7],
        cb0, cb1, cb2)
    recon_mean = recon_sum / BATCH
    rq_mean = rq_sum / BATCH
    loss = recon_mean + 3.0 * rq_mean
    return loss, recon_mean, rq_mean


# speedup vs baseline: 1.2070x; 1.0450x over previous
import functools

import jax
import jax.numpy as jnp
from jax.experimental import pallas as pl
from jax.experimental.pallas import tpu as pltpu

BATCH = 4096
TILE = 1024
HALF = 512
INPUT_DIM = 768
EMBED_DIM = 64
CODEBOOK_SIZE = 1024
N_LAYERS = 3
N_CAT = 18
COMMIT_W = 0.25


def _silu(v):
    return 0.5 * v * (1.0 + jnp.tanh(0.5 * v))


def _dot(a, b):
    return jax.lax.dot_general(a, b, (((1,), (0,)), ((), ())),
                               preferred_element_type=jnp.float32)


def _half_losses(xb, enc, dec, cbs):
    (ew0, eb0, ew1, eb1, ew2, eb2, ew3, eb3) = enc
    (dw0, db0, dw1, db1, dw2, db2, dw3, db3) = dec

    h = _silu(_dot(xb, ew0) + eb0)
    h = _silu(_dot(h, ew1) + eb1)
    h = _silu(_dot(h, ew2) + eb2)
    h = _dot(h, ew3) + eb3
    n = jnp.sqrt(jnp.sum(h * h, axis=-1, keepdims=True))
    res = h / jnp.maximum(n, 1e-12)

    emb_sum = jnp.zeros((HALF, EMBED_DIM), dtype=jnp.float32)
    rq_row = jnp.zeros((HALF, 1), dtype=jnp.float32)

    for cb, half_cb_sq in cbs:
        t = _dot(res, cb.T) - half_cb_sq
        tmax = jnp.max(t, axis=-1, keepdims=True)
        onehot = (t == tmax).astype(jnp.float32)
        emb = _dot(onehot, cb)
        rn = jnp.sqrt(jnp.sum(res * res, axis=-1, keepdims=True))
        u = res / (rn + 1e-8)
        qn = jnp.sqrt(jnp.sum(emb * emb, axis=-1, keepdims=True))
        q = emb / (qn + 1e-8)
        w = u + q
        wn = jnp.sqrt(jnp.sum(w * w, axis=-1, keepdims=True))
        w = w / jnp.maximum(wn, 1e-6)
        ew = jnp.sum(res * w, axis=-1, keepdims=True)
        eu = jnp.sum(res * u, axis=-1, keepdims=True)
        out = res - 2.0 * ew * w + 2.0 * eu * q
        new_res = res - out
        rq_row = rq_row + jnp.sum(new_res * new_res, axis=-1, keepdims=True)
        emb_sum = emb_sum + out
        res = new_res

    h = _silu(_dot(emb_sum, dw0) + db0)
    h = _silu(_dot(h, dw1) + db1)
    h = _silu(_dot(h, dw2) + db2)
    x_hat = _dot(h, dw3) + db3

    colx = jax.lax.broadcasted_iota(jnp.int32, (HALF, INPUT_DIM), 1)
    is_mse = colx < (INPUT_DIM - N_CAT)
    diff = x_hat - xb
    mse_row = jnp.sum(jnp.where(is_mse, diff * diff, 0.0), axis=-1)
    lg = x_hat[:, INPUT_DIM - 128:]
    tg = xb[:, INPUT_DIM - 128:]
    colb = jax.lax.broadcasted_iota(jnp.int32, (HALF, 128), 1)
    bce = (jnp.maximum(lg, 0.0) - lg * tg
           + jnp.log1p(jnp.exp(-jnp.abs(lg))))
    bce_row = jnp.sum(jnp.where(colb >= 128 - N_CAT, bce, 0.0), axis=-1)
    recon_sum = jnp.sum(mse_row + bce_row)
    rq_sum = (1.0 + COMMIT_W) * jnp.sum(rq_row)
    return recon_sum, rq_sum


def _fused_body(x_ref,
                ew0, eb0, ew1, eb1, ew2, eb2, ew3, eb3,
                dw0, db0, dw1, db1, dw2, db2, dw3, db3,
                cb0, cb1, cb2,
                recon_ref, rq_ref):
    i = pl.program_id(0)
    enc = (ew0[...], eb0[...], ew1[...], eb1[...],
           ew2[...], eb2[...], ew3[...], eb3[...])
    dec = (dw0[...], db0[...], dw1[...], db1[...],
           dw2[...], db2[...], dw3[...], db3[...])
    cbs = []
    for cb_ref in (cb0, cb1, cb2):
        cbm = cb_ref[...]
        cbn = jnp.sqrt(jnp.sum(cbm * cbm, axis=-1, keepdims=True))
        cb = cbm / jnp.maximum(cbn, 1e-12)
        cbs.append((cb, 0.5 * jnp.sum(cb * cb, axis=-1)[None, :]))

    recon0, rq0 = _half_losses(x_ref[0:HALF, :], enc, dec, cbs)
    recon1, rq1 = _half_losses(x_ref[HALF:TILE, :], enc, dec, cbs)
    recon_sum = recon0 + recon1
    rq_sum = rq0 + rq1

    @pl.when(i == 0)
    def _init():
        recon_ref[...] = jnp.zeros((1, 1), jnp.float32)
        rq_ref[...] = jnp.zeros((1, 1), jnp.float32)

    recon_ref[...] += recon_sum.reshape(1, 1)
    rq_ref[...] += rq_sum.reshape(1, 1)


@functools.partial(jax.jit, static_argnames=())
def _fused(x, ew0, eb0, ew1, eb1, ew2, eb2, ew3, eb3,
           dw0, db0, dw1, db1, dw2, db2, dw3, db3, cb0, cb1, cb2):
    num_tiles = BATCH // TILE

    def wspec(shape):
        return pl.BlockSpec(shape, lambda i: (0,) * len(shape))

    in_specs = [pl.BlockSpec((TILE, INPUT_DIM), lambda i: (i, 0))]
    ws = [ew0, eb0, ew1, eb1, ew2, eb2, ew3, eb3,
          dw0, db0, dw1, db1, dw2, db2, dw3, db3, cb0, cb1, cb2]
    in_specs += [wspec(w.shape) for w in ws]

    out_shape = (jax.ShapeDtypeStruct((1, 1), jnp.float32),
                 jax.ShapeDtypeStruct((1, 1), jnp.float32))
    out_specs = (pl.BlockSpec((1, 1), lambda i: (0, 0)),
                 pl.BlockSpec((1, 1), lambda i: (0, 0)))

    recon_sum, rq_sum = pl.pallas_call(
        _fused_body,
        grid=(num_tiles,),
        in_specs=in_specs,
        out_specs=out_specs,
        out_shape=out_shape,
        compiler_params=pltpu.CompilerParams(
            dimension_semantics=("arbitrary",),
            vmem_limit_bytes=128 * 1024 * 1024,
        ),
    )(x, *ws)
    return recon_sum[0, 0], rq_sum[0, 0]


def kernel(x, enc_w0, enc_b0, enc_w1, enc_b1, enc_w2, enc_b2, enc_w3, enc_b3,
           dec_w0, dec_b0, dec_w1, dec_b1, dec_w2, dec_b2, dec_w3, dec_b3,
           cb0, cb1, cb2, gumbel_t):
    del gumbel_t
    bs = [b.reshape(1, -1) for b in
          (enc_b0, enc_b1, enc_b2, enc_b3, dec_b0, dec_b1, dec_b2, dec_b3)]
    ws = [enc_w0, enc_w1, enc_w2, enc_w3, dec_w0, dec_w1, dec_w2, dec_w3]
    recon_sum, rq_sum = _fused(
        x, ws[0], bs[0], ws[1], bs[1], ws[2], bs[2], ws[3], bs[3],
        ws[4], bs[4], ws[5], bs[5], ws[6], bs[6], ws[7], bs[7],
        cb0, cb1, cb2)
    recon_mean = recon_sum / BATCH
    rq_mean = rq_sum / BATCH
    loss = recon_mean + 3.0 * rq_mean
    return loss, recon_mean, rq_mean
